# baseline (speedup 1.0000x reference)
import functools

import jax
import jax.numpy as jnp
from jax import lax
from jax.experimental import pallas as pl
from jax.experimental.pallas import tpu as pltpu
from jax.experimental.pallas import tpu_sc as plsc

_LANES = 16


def _sc_encoder(batch, num_patches, proj_dim, dtype):
    info = plsc.get_sparse_core_info()
    n_workers = info.num_cores * info.num_subcores
    rows_per_w = num_patches // n_workers

    mesh = plsc.VectorSubcoreMesh(core_axis_name="c", subcore_axis_name="s")

    @functools.partial(
        pl.kernel,
        mesh=mesh,
        out_type=jax.ShapeDtypeStruct((batch, num_patches, proj_dim), dtype),
        scratch_types=[
            pltpu.VMEM((rows_per_w, proj_dim), dtype),
            pltpu.VMEM((rows_per_w, proj_dim), dtype),
        ],
    )
    def k(patch_hbm, pos_hbm, out_hbm, pos_v, buf_v):
        wid = lax.axis_index("s") * info.num_cores + lax.axis_index("c")
        base = wid * rows_per_w
        pltpu.sync_copy(pos_hbm.at[pl.ds(base, rows_per_w)], pos_v)

        def per_batch(b, carry):
            pltpu.sync_copy(patch_hbm.at[b, pl.ds(base, rows_per_w)], buf_v)

            def per_row(i, c2):
                for j in range(proj_dim // _LANES):
                    sl = pl.ds(j * _LANES, _LANES)
                    buf_v[i, sl] = buf_v[i, sl] + pos_v[i, sl]
                return c2

            lax.fori_loop(0, rows_per_w, per_row, 0, unroll=False)
            pltpu.sync_copy(buf_v, out_hbm.at[b, pl.ds(base, rows_per_w)])
            return carry

        lax.fori_loop(0, batch, per_batch, 0, unroll=False)

    return k


def kernel(patch, pos_table):
    batch, num_patches, proj_dim = patch.shape
    return _sc_encoder(batch, num_patches, proj_dim, patch.dtype)(
        patch, pos_table
    )


# speedup vs baseline: 1.9465x; 1.9465x over previous
import functools

import jax
import jax.numpy as jnp
from jax import lax
from jax.experimental import pallas as pl
from jax.experimental.pallas import tpu as pltpu
from jax.experimental.pallas import tpu_sc as plsc

_LANES = 16
_NBUF = 2


def _sc_encoder(batch, num_patches, proj_dim, dtype):
    info = plsc.get_sparse_core_info()
    n_workers = info.num_cores * info.num_subcores
    rows_per_w = num_patches // n_workers

    mesh = plsc.VectorSubcoreMesh(core_axis_name="c", subcore_axis_name="s")

    @functools.partial(
        pl.kernel,
        mesh=mesh,
        out_type=jax.ShapeDtypeStruct((batch, num_patches, proj_dim), dtype),
        scratch_types=[
            pltpu.VMEM((rows_per_w, proj_dim), dtype),
            [pltpu.VMEM((rows_per_w, proj_dim), dtype) for _ in range(_NBUF)],
            [pltpu.VMEM((rows_per_w, proj_dim), dtype) for _ in range(_NBUF)],
            [pltpu.SemaphoreType.DMA for _ in range(_NBUF)],
            [pltpu.SemaphoreType.DMA for _ in range(_NBUF)],
        ],
    )
    def k(patch_hbm, pos_hbm, out_hbm, pos_v, in_v, out_v, in_sem, out_sem):
        wid = lax.axis_index("s") * info.num_cores + lax.axis_index("c")
        base = wid * rows_per_w
        rows = pl.ds(base, rows_per_w)
        pltpu.sync_copy(pos_hbm.at[rows], pos_v)

        for b0 in range(_NBUF):
            pltpu.async_copy(patch_hbm.at[b0, rows], in_v[b0], in_sem[b0])

        def per_pair(pair, carry):
            for s in range(_NBUF):
                b = pair * _NBUF + s
                pltpu.make_async_copy(
                    patch_hbm.at[b, rows], in_v[s], in_sem[s]
                ).wait()

                @pl.when(b >= _NBUF)
                def _():
                    pltpu.make_async_copy(
                        out_v[s], out_hbm.at[b - _NBUF, rows], out_sem[s]
                    ).wait()

                def per_row(i, c2):
                    for j in range(proj_dim // _LANES):
                        sl = pl.ds(j * _LANES, _LANES)
                        out_v[s][i, sl] = in_v[s][i, sl] + pos_v[i, sl]
                    return c2

                lax.fori_loop(0, rows_per_w, per_row, 0, unroll=False)
                pltpu.async_copy(out_v[s], out_hbm.at[b, rows], out_sem[s])

                @pl.when(b + _NBUF < batch)
                def _():
                    pltpu.async_copy(
                        patch_hbm.at[b + _NBUF, rows], in_v[s], in_sem[s]
                    )

            return carry

        lax.fori_loop(0, batch // _NBUF, per_pair, 0, unroll=False)

        for s in range(_NBUF):
            pltpu.make_async_copy(
                out_v[s], out_hbm.at[batch - _NBUF + s, rows], out_sem[s]
            ).wait()

    return k


def kernel(patch, pos_table):
    batch, num_patches, proj_dim = patch.shape
    return _sc_encoder(batch, num_patches, proj_dim, patch.dtype)(
        patch, pos_table
    )
